# baseline (speedup 1.0000x reference)
import functools

import jax
import jax.numpy as jnp
from jax import lax
from jax.experimental import pallas as pl
from jax.experimental.pallas import tpu as pltpu
from jax.experimental.pallas import tpu_sc as plsc

_INDIM = 2048
_OUTDIM = 2048
_FANIN = 32
_NTOK = 2048

_NUM_WORKERS = 32
_ROWS_PER_WORKER = _OUTDIM // _NUM_WORKERS
_CHUNK_ROWS = 16
_CHUNK_EDGES = _CHUNK_ROWS * _FANIN
_LANES = 16
_NBUF = 2


def _scatter_body(srcp_hbm, wp_hbm, wout_hbm, wbufs, srcbuf, wvbuf, sems):
    wid = lax.axis_index("s") * 2 + lax.axis_index("c")
    iota = lax.iota(jnp.int32, _LANES)
    zeros16 = jnp.zeros((_LANES,), jnp.float32)
    nchunks = _ROWS_PER_WORKER // _CHUNK_ROWS
    pending = [None] * _NBUF

    for buf in range(_NBUF):
        for r in range(_CHUNK_ROWS):
            def _zcol(j, carry, buf=buf, r=r):
                base = j * (_LANES * 8)
                for u in range(8):
                    wbufs[buf, r, pl.ds(base + u * _LANES, _LANES)] = zeros16
                return carry
            lax.fori_loop(0, _INDIM // (_LANES * 8), _zcol, 0)

    for chunk in range(nchunks):
        buf = chunk % _NBUF
        row_base = wid * _ROWS_PER_WORKER + chunk * _CHUNK_ROWS
        cidx = row_base // _CHUNK_ROWS
        wbuf = wbufs.at[buf]

        if pending[buf] is not None:
            pending[buf].wait()
            pending[buf] = None
            for h in range(0, _CHUNK_ROWS, _LANES):
                r_loc = iota + h
                for k in range(_FANIN):
                    src_vec = srcbuf[buf, k, pl.ds(h, _LANES)]
                    w_vec = wvbuf[buf, k, pl.ds(h, _LANES)]
                    plsc.addupdate_scatter(wbuf, [r_loc, src_vec], -w_vec)

        pltpu.sync_copy(srcp_hbm.at[cidx], srcbuf.at[buf])
        pltpu.sync_copy(wp_hbm.at[cidx], wvbuf.at[buf])

        for h in range(0, _CHUNK_ROWS, _LANES):
            r_loc = iota + h
            for k in range(_FANIN):
                src_vec = srcbuf[buf, k, pl.ds(h, _LANES)]
                w_vec = wvbuf[buf, k, pl.ds(h, _LANES)]
                plsc.addupdate_scatter(wbuf, [r_loc, src_vec], w_vec)

        pending[buf] = pltpu.async_copy(
            wbuf, wout_hbm.at[pl.ds(row_base, _CHUNK_ROWS)], sems.at[buf])

    for p in pending:
        if p is not None:
            p.wait()


_NUM_CHUNKS = _OUTDIM // _CHUNK_ROWS


def _build_w(src_p, weight_p):
    mesh = plsc.VectorSubcoreMesh(core_axis_name="c", subcore_axis_name="s")
    k = pl.kernel(
        _scatter_body,
        mesh=mesh,
        out_type=jax.ShapeDtypeStruct((_OUTDIM, _INDIM), jnp.float32),
        scratch_types=[
            pltpu.VMEM((_NBUF, _CHUNK_ROWS, _INDIM), jnp.float32),
            pltpu.VMEM((_NBUF, _FANIN, _CHUNK_ROWS), jnp.int32),
            pltpu.VMEM((_NBUF, _FANIN, _CHUNK_ROWS), jnp.float32),
            pltpu.SemaphoreType.DMA((_NBUF,)),
        ],
        compiler_params=pltpu.CompilerParams(needs_layout_passes=False),
    )
    return k(src_p, weight_p)


_BN = 512


def _mm_body(x_ref, w_ref, b_ref, o_ref):
    acc = lax.dot_general(
        x_ref[...], w_ref[...], (((1,), (1,)), ((), ())),
        preferred_element_type=jnp.float32,
    )
    o_ref[...] = acc + b_ref[...]


def _matmul(x, w, bias):
    return pl.pallas_call(
        _mm_body,
        grid=(_OUTDIM // _BN,),
        in_specs=[
            pl.BlockSpec((_NTOK, _INDIM), lambda j: (0, 0)),
            pl.BlockSpec((_BN, _INDIM), lambda j: (j, 0)),
            pl.BlockSpec((1, _BN), lambda j: (0, j)),
        ],
        out_specs=pl.BlockSpec((_NTOK, _BN), lambda j: (0, j)),
        out_shape=jax.ShapeDtypeStruct((_NTOK, _OUTDIM), jnp.float32),
    )(x, w, bias.reshape(1, _OUTDIM))


@jax.jit
def kernel(x, weight, bias, edge_index):
    src_p = edge_index[1].reshape(_NUM_CHUNKS, _CHUNK_ROWS, _FANIN)
    src_p = src_p.transpose(0, 2, 1)
    weight_p = weight.reshape(_NUM_CHUNKS, _CHUNK_ROWS, _FANIN)
    weight_p = weight_p.transpose(0, 2, 1)
    w = _build_w(src_p, weight_p)
    return _matmul(x, w, bias)


# speedup vs baseline: 1.0334x; 1.0334x over previous
import functools

import jax
import jax.numpy as jnp
from jax import lax
from jax.experimental import pallas as pl
from jax.experimental.pallas import tpu as pltpu
from jax.experimental.pallas import tpu_sc as plsc

_INDIM = 2048
_OUTDIM = 2048
_FANIN = 32
_NTOK = 2048

_NUM_WORKERS = 32
_ROWS_PER_WORKER = _OUTDIM // _NUM_WORKERS
_CHUNK_ROWS = 16
_CHUNK_EDGES = _CHUNK_ROWS * _FANIN
_LANES = 16
_NBUF = 2


def _scatter_body(src_hbm, w_hbm, wout_hbm, wbufs, srcbuf, wvbuf, sems):
    wid = lax.axis_index("s") * 2 + lax.axis_index("c")
    iota = lax.iota(jnp.int32, _LANES)
    zeros16 = jnp.zeros((_LANES,), jnp.float32)
    nchunks = _ROWS_PER_WORKER // _CHUNK_ROWS
    pending = [None] * _NBUF

    for buf in range(_NBUF):
        for r in range(_CHUNK_ROWS):
            def _zcol(j, carry, buf=buf, r=r):
                base = j * (_LANES * 8)
                for u in range(8):
                    wbufs[buf, r, pl.ds(base + u * _LANES, _LANES)] = zeros16
                return carry
            lax.fori_loop(0, _INDIM // (_LANES * 8), _zcol, 0)

    for chunk in range(nchunks):
        buf = chunk % _NBUF
        row_base = wid * _ROWS_PER_WORKER + chunk * _CHUNK_ROWS
        edge_base = row_base * _FANIN
        wbuf = wbufs.at[buf]

        if pending[buf] is not None:
            pending[buf].wait()
            pending[buf] = None
            for k in range(_FANIN):
                le = iota * _FANIN + (buf * _CHUNK_EDGES + k)
                src_vec = plsc.load_gather(srcbuf, [le])
                w_vec = plsc.load_gather(wvbuf, [le])
                plsc.addupdate_scatter(wbuf, [iota, src_vec], -w_vec)

        pltpu.sync_copy(src_hbm.at[pl.ds(edge_base, _CHUNK_EDGES)],
                        srcbuf.at[pl.ds(buf * _CHUNK_EDGES, _CHUNK_EDGES)])
        pltpu.sync_copy(w_hbm.at[pl.ds(edge_base, _CHUNK_EDGES)],
                        wvbuf.at[pl.ds(buf * _CHUNK_EDGES, _CHUNK_EDGES)])

        for k in range(_FANIN):
            le = iota * _FANIN + (buf * _CHUNK_EDGES + k)
            src_vec = plsc.load_gather(srcbuf, [le])
            w_vec = plsc.load_gather(wvbuf, [le])
            plsc.addupdate_scatter(wbuf, [iota, src_vec], w_vec)

        pending[buf] = pltpu.async_copy(
            wbuf, wout_hbm.at[pl.ds(row_base, _CHUNK_ROWS)], sems.at[buf])

    for p in pending:
        if p is not None:
            p.wait()


_NUM_CHUNKS = _OUTDIM // _CHUNK_ROWS


def _build_w(src, weight):
    mesh = plsc.VectorSubcoreMesh(core_axis_name="c", subcore_axis_name="s")
    k = pl.kernel(
        _scatter_body,
        mesh=mesh,
        out_type=jax.ShapeDtypeStruct((_OUTDIM, _INDIM), jnp.float32),
        scratch_types=[
            pltpu.VMEM((_NBUF, _CHUNK_ROWS, _INDIM), jnp.float32),
            pltpu.VMEM((_NBUF * _CHUNK_EDGES,), jnp.int32),
            pltpu.VMEM((_NBUF * _CHUNK_EDGES,), jnp.float32),
            pltpu.SemaphoreType.DMA((_NBUF,)),
        ],
        compiler_params=pltpu.CompilerParams(needs_layout_passes=False),
    )
    return k(src, weight)


_BN = 512


def _mm_body(x_ref, w_ref, b_ref, o_ref):
    acc = lax.dot_general(
        x_ref[...], w_ref[...], (((1,), (1,)), ((), ())),
        preferred_element_type=jnp.float32,
    )
    o_ref[...] = acc + b_ref[...]


def _matmul(x, w, bias):
    return pl.pallas_call(
        _mm_body,
        grid=(_OUTDIM // _BN,),
        in_specs=[
            pl.BlockSpec((_NTOK, _INDIM), lambda j: (0, 0)),
            pl.BlockSpec((_BN, _INDIM), lambda j: (j, 0)),
            pl.BlockSpec((1, _BN), lambda j: (0, j)),
        ],
        out_specs=pl.BlockSpec((_NTOK, _BN), lambda j: (0, j)),
        out_shape=jax.ShapeDtypeStruct((_NTOK, _OUTDIM), jnp.float32),
    )(x, w, bias.reshape(1, _OUTDIM))


@jax.jit
def kernel(x, weight, bias, edge_index):
    w = _build_w(edge_index[1], weight)
    return _matmul(x, w, bias)
